# Initial kernel scaffold; baseline (speedup 1.0000x reference)
import jax
import jax.numpy as jnp
from jax.experimental import pallas as pl


def kernel(features, node_order, adjacency_list, edge_order, pe, emb_table, sort_table, W_iou, b_iou, U_iou, W_f, b_f, U_f):
    raise NotImplementedError("write your pallas kernel here")


# speedup vs baseline: 26.6685x; 26.6685x over previous
import functools
import numpy as np
import jax
import jax.numpy as jnp
from jax import lax
from jax.experimental import pallas as pl
from jax.experimental.pallas import tpu as pltpu
from jax.experimental.pallas import tpu_sc as plsc

_DEPTH = 9
_NT = 64
_TD = 10
_NPT = 2 ** (_DEPTH + 1) - 1
_N = _NT * _NPT
_PAD_N = 65536
_NC, _NS = 2, 16
_NW = _NC * _NS
_ROWS_PER_W = _PAD_N // _NW
_CH = 1024


def _build_perm():
    sizes = [2 ** (_DEPTH - l) for l in range(_DEPTH + 1)]
    offs = np.concatenate([[0], np.cumsum(sizes)])
    tree = np.arange(_NT, dtype=np.int64)
    pos = np.zeros(_NT, dtype=np.int64)
    blocks = [None] * (_DEPTH + 1)
    blocks[_DEPTH] = tree * _NPT + offs[_DEPTH] + pos
    for l in range(_DEPTH, 0, -1):
        tree = np.concatenate([tree, tree])
        pos = np.concatenate([2 * pos, 2 * pos + 1])
        blocks[l - 1] = tree * _NPT + offs[l - 1] + pos
    perm = np.concatenate([blocks[l] for l in range(_DEPTH + 1)])
    perm = np.concatenate([perm, np.zeros(_PAD_N - _N, dtype=np.int64)])
    return perm.astype(np.int32)


_PERM = _build_perm()
_LV_N = [_NT * 2 ** (_DEPTH - l) for l in range(_DEPTH + 1)]


def _sc_gather_body(tok_hbm, sort_hbm, pos_hbm, emb_hbm, stab_hbm, pe_hbm,
                    otok, osort, ope, idx_v, r16_v, r32_v, sem):
    wid = lax.axis_index("s") * _NC + lax.axis_index("c")
    for k in range(_ROWS_PER_W // _CH):
        b = wid * _ROWS_PER_W + k * _CH
        rows = pl.ds(b, _CH)
        pltpu.sync_copy(tok_hbm.at[rows], idx_v)
        pltpu.async_copy(emb_hbm.at[idx_v], r16_v, sem).wait()
        pltpu.sync_copy(r16_v, otok.at[rows])
        pltpu.sync_copy(sort_hbm.at[rows], idx_v)
        pltpu.async_copy(stab_hbm.at[idx_v], r16_v, sem).wait()
        pltpu.sync_copy(r16_v, osort.at[rows])
        pltpu.sync_copy(pos_hbm.at[rows], idx_v)
        pltpu.async_copy(pe_hbm.at[idx_v], r32_v, sem).wait()
        pltpu.sync_copy(r32_v, ope.at[rows])


@functools.cache
def _make_sc_gather():
    return functools.partial(
        pl.kernel,
        out_type=[
            jax.ShapeDtypeStruct((_PAD_N, 16), jnp.float32),
            jax.ShapeDtypeStruct((_PAD_N, 16), jnp.float32),
            jax.ShapeDtypeStruct((_PAD_N, 32), jnp.float32),
        ],
        mesh=plsc.VectorSubcoreMesh(core_axis_name="c", subcore_axis_name="s"),
        scratch_types=[
            pltpu.VMEM((_CH,), jnp.int32),
            pltpu.VMEM((_CH, 16), jnp.float32),
            pltpu.VMEM((_CH, 32), jnp.float32),
            pltpu.SemaphoreType.DMA,
        ],
        compiler_params=pltpu.CompilerParams(use_tc_tiling_on_sc=False),
    )(_sc_gather_body)


def _sig(x):
    return jax.nn.sigmoid(x)


_PROJ_CH = 4096


def _tc_proj_body(tok_ref, sort_ref, pe_ref, wt_ref, b_ref, out_ref):
    x = jnp.concatenate([tok_ref[:], sort_ref[:], pe_ref[:]], axis=1)
    res_t = jax.lax.dot_general(
        wt_ref[:], x, (((1,), (1,)), ((), ())),
        preferred_element_type=jnp.float32)
    out_ref[:] = res_t + b_ref[:]


def _tc_walk_body(iouf_ref, uit_ref, uot_ref, uut_ref, uft_ref, out_ref):
    n = _LV_N[0]
    i = _sig(iouf_ref[0:16, 0:n])
    o = _sig(iouf_ref[16:32, 0:n])
    u = jnp.tanh(iouf_ref[32:48, 0:n])
    c = i * u
    h = o * jnp.tanh(c)
    off = n
    for l in range(1, _DEPTH + 1):
        n = _LV_N[l]
        hs = h[:, :n] + h[:, n:]
        i = _sig(iouf_ref[0:16, off:off + n] +
                 jnp.dot(uit_ref[:], hs, preferred_element_type=jnp.float32))
        o = _sig(iouf_ref[16:32, off:off + n] +
                 jnp.dot(uot_ref[:], hs, preferred_element_type=jnp.float32))
        u = jnp.tanh(iouf_ref[32:48, off:off + n] +
                     jnp.dot(uut_ref[:], hs,
                             preferred_element_type=jnp.float32))
        fp = iouf_ref[48:64, off:off + n]
        f = _sig(jnp.concatenate([fp, fp], axis=1) +
                 jnp.dot(uft_ref[:], h, preferred_element_type=jnp.float32))
        fc = f * c
        c = i * u + fc[:, :n] + fc[:, n:]
        h = o * jnp.tanh(c)
        off += n
    out_ref[:] = h


def _spread_rows(w):
    out = jnp.zeros((64, w.shape[1]), jnp.float32)
    out = out.at[0:10].set(w[0:10])
    out = out.at[16:26].set(w[10:20])
    out = out.at[32:64].set(w[20:52])
    return out


def _pad16(m):
    return jnp.pad(m, ((0, 16 - m.shape[0]), (0, 16 - m.shape[1])))


def kernel(features, node_order, adjacency_list, edge_order, pe, emb_table,
           sort_table, W_iou, b_iou, U_iou, W_f, b_f, U_f):
    perm = jnp.asarray(_PERM)
    tok_ids = features[:, 0][perm]
    sort_ids = features[:, 1][perm]
    pos_ids = features[:, 2][perm]
    emb16 = jnp.pad(emb_table, ((0, 0), (0, 6)))
    stab16 = jnp.pad(sort_table, ((0, 0), (0, 6)))
    tokf, sortf, pef = _make_sc_gather()(tok_ids, sort_ids, pos_ids,
                                         emb16, stab16, pe)
    gates = [W_iou[:, 0:10], W_iou[:, 10:20], W_iou[:, 20:30], W_f]
    biases = [b_iou[0:10], b_iou[10:20], b_iou[20:30], b_f]
    w_all = jnp.concatenate(
        [jnp.pad(_spread_rows(g), ((0, 0), (0, 6))) for g in gates], axis=1)
    b_all = jnp.concatenate([jnp.pad(b, (0, 6)) for b in biases])
    iouf_t = pl.pallas_call(
        _tc_proj_body,
        grid=(_PAD_N // _PROJ_CH,),
        in_specs=[
            pl.BlockSpec((_PROJ_CH, 16), lambda g: (g, 0)),
            pl.BlockSpec((_PROJ_CH, 16), lambda g: (g, 0)),
            pl.BlockSpec((_PROJ_CH, 32), lambda g: (g, 0)),
            pl.BlockSpec((64, 64), lambda g: (0, 0)),
            pl.BlockSpec((64, 1), lambda g: (0, 0)),
        ],
        out_specs=pl.BlockSpec((64, _PROJ_CH), lambda g: (0, g)),
        out_shape=jax.ShapeDtypeStruct((64, _PAD_N), jnp.float32),
    )(tokf, sortf, pef, w_all.T, b_all.reshape(64, 1))
    u_gates = [U_iou[:, 0:10], U_iou[:, 10:20], U_iou[:, 20:30], U_f]
    uit, uot, uut, uft = [_pad16(u.T) for u in u_gates]
    h_roots_t = pl.pallas_call(
        _tc_walk_body,
        out_shape=jax.ShapeDtypeStruct((16, _NT), jnp.float32),
    )(iouf_t, uit, uot, uut, uft)
    return h_roots_t[0:_TD].T


# speedup vs baseline: 28.7888x; 1.0795x over previous
import functools
import numpy as np
import jax
import jax.numpy as jnp
from jax import lax
from jax.experimental import pallas as pl
from jax.experimental.pallas import tpu as pltpu
from jax.experimental.pallas import tpu_sc as plsc

_DEPTH = 9
_NT = 64
_TD = 10
_NPT = 2 ** (_DEPTH + 1) - 1
_N = _NT * _NPT
_PAD_N = 65536
_NC, _NS = 2, 16
_NW = _NC * _NS
_ROWS_PER_W = _PAD_N // _NW
_CH = 1024


def _build_perm():
    sizes = [2 ** (_DEPTH - l) for l in range(_DEPTH + 1)]
    offs = np.concatenate([[0], np.cumsum(sizes)])
    tree = np.arange(_NT, dtype=np.int64)
    pos = np.zeros(_NT, dtype=np.int64)
    blocks = [None] * (_DEPTH + 1)
    blocks[_DEPTH] = tree * _NPT + offs[_DEPTH] + pos
    for l in range(_DEPTH, 0, -1):
        tree = np.concatenate([tree, tree])
        pos = np.concatenate([2 * pos, 2 * pos + 1])
        blocks[l - 1] = tree * _NPT + offs[l - 1] + pos
    perm = np.concatenate([blocks[l] for l in range(_DEPTH + 1)])
    perm = np.concatenate([perm, np.zeros(_PAD_N - _N, dtype=np.int64)])
    return perm.astype(np.int32)


_PERM = _build_perm()
_LV_N = [_NT * 2 ** (_DEPTH - l) for l in range(_DEPTH + 1)]


def _sc_gather_body(perm_hbm, tok_hbm, sort_hbm, pos_hbm,
                    emb_hbm, stab_hbm, pe_hbm,
                    otok, osort, ope, pidx_v, idx_v, r16_v, r32_v, sem):
    wid = lax.axis_index("s") * _NC + lax.axis_index("c")
    for k in range(_ROWS_PER_W // _CH):
        b = wid * _ROWS_PER_W + k * _CH
        rows = pl.ds(b, _CH)
        pltpu.sync_copy(perm_hbm.at[rows], pidx_v)
        pltpu.async_copy(tok_hbm.at[pidx_v], idx_v, sem).wait()
        pltpu.async_copy(emb_hbm.at[idx_v], r16_v, sem).wait()
        pltpu.sync_copy(r16_v, otok.at[rows])
        pltpu.async_copy(sort_hbm.at[pidx_v], idx_v, sem).wait()
        pltpu.async_copy(stab_hbm.at[idx_v], r16_v, sem).wait()
        pltpu.sync_copy(r16_v, osort.at[rows])
        pltpu.async_copy(pos_hbm.at[pidx_v], idx_v, sem).wait()
        pltpu.async_copy(pe_hbm.at[idx_v], r32_v, sem).wait()
        pltpu.sync_copy(r32_v, ope.at[rows])


@functools.cache
def _make_sc_gather():
    return functools.partial(
        pl.kernel,
        out_type=[
            jax.ShapeDtypeStruct((_PAD_N, 16), jnp.float32),
            jax.ShapeDtypeStruct((_PAD_N, 16), jnp.float32),
            jax.ShapeDtypeStruct((_PAD_N, 32), jnp.float32),
        ],
        mesh=plsc.VectorSubcoreMesh(core_axis_name="c", subcore_axis_name="s"),
        scratch_types=[
            pltpu.VMEM((_CH,), jnp.int32),
            pltpu.VMEM((_CH,), jnp.int32),
            pltpu.VMEM((_CH, 16), jnp.float32),
            pltpu.VMEM((_CH, 32), jnp.float32),
            pltpu.SemaphoreType.DMA,
        ],
        compiler_params=pltpu.CompilerParams(use_tc_tiling_on_sc=False),
    )(_sc_gather_body)


def _sig(x):
    return jax.nn.sigmoid(x)


_PROJ_CH = 4096
_NPROJ = _PAD_N // _PROJ_CH


def _tc_body(tok_ref, sort_ref, pe_ref, wt_ref, b_ref,
             uit_ref, uot_ref, uut_ref, uft_ref, out_ref, iouf_ref):
    g = pl.program_id(0)

    @pl.when(g < _NPROJ)
    def _proj():
        x = jnp.concatenate([tok_ref[:, 0:_TD], sort_ref[:, 0:_TD],
                             pe_ref[:]], axis=1)
        res_t = jax.lax.dot_general(
            wt_ref[:], x, (((1,), (1,)), ((), ())),
            preferred_element_type=jnp.float32)
        iouf_ref[:, pl.ds(g * _PROJ_CH, _PROJ_CH)] = res_t + b_ref[:]

    @pl.when(g == _NPROJ)
    def _walk():
        n = _LV_N[0]
        i = _sig(iouf_ref[0:16, 0:n])
        o = _sig(iouf_ref[16:32, 0:n])
        u = jnp.tanh(iouf_ref[32:48, 0:n])
        c = i * u
        h = o * jnp.tanh(c)
        off = n
        for l in range(1, _DEPTH + 1):
            n = _LV_N[l]
            hs = h[:, :n] + h[:, n:]
            i = _sig(iouf_ref[0:16, off:off + n] +
                     jnp.dot(uit_ref[:], hs,
                             preferred_element_type=jnp.float32))
            o = _sig(iouf_ref[16:32, off:off + n] +
                     jnp.dot(uot_ref[:], hs,
                             preferred_element_type=jnp.float32))
            u = jnp.tanh(iouf_ref[32:48, off:off + n] +
                         jnp.dot(uut_ref[:], hs,
                                 preferred_element_type=jnp.float32))
            fp = iouf_ref[48:64, off:off + n]
            f = _sig(jnp.concatenate([fp, fp], axis=1) +
                     jnp.dot(uft_ref[:], h,
                             preferred_element_type=jnp.float32))
            fc = f * c
            c = i * u + fc[:, :n] + fc[:, n:]
            h = o * jnp.tanh(c)
            off += n
        out_ref[:] = h


def _pad16(m):
    return jnp.pad(m, ((0, 16 - m.shape[0]), (0, 16 - m.shape[1])))


def _tc_call(tokf, sortf, pef, w4t, b4, uit, uot, uut, uft, interpret=False):
    return pl.pallas_call(
        _tc_body,
        grid=(_NPROJ + 1,),
        in_specs=[
            pl.BlockSpec((_PROJ_CH, 16),
                         lambda g: (jnp.minimum(g, _NPROJ - 1), 0)),
            pl.BlockSpec((_PROJ_CH, 16),
                         lambda g: (jnp.minimum(g, _NPROJ - 1), 0)),
            pl.BlockSpec((_PROJ_CH, 32),
                         lambda g: (jnp.minimum(g, _NPROJ - 1), 0)),
            pl.BlockSpec((64, 52), lambda g: (0, 0)),
            pl.BlockSpec((64, 1), lambda g: (0, 0)),
            pl.BlockSpec((16, 16), lambda g: (0, 0)),
            pl.BlockSpec((16, 16), lambda g: (0, 0)),
            pl.BlockSpec((16, 16), lambda g: (0, 0)),
            pl.BlockSpec((16, 16), lambda g: (0, 0)),
        ],
        out_specs=pl.BlockSpec((16, _NT), lambda g: (0, 0)),
        out_shape=jax.ShapeDtypeStruct((16, _NT), jnp.float32),
        scratch_shapes=[pltpu.VMEM((64, _PAD_N), jnp.float32)],
        interpret=interpret,
    )(tokf, sortf, pef, w4t, b4, uit, uot, uut, uft)


def _prep_weights(W_iou, b_iou, U_iou, W_f, b_f, U_f):
    gates = [W_iou[:, 0:10], W_iou[:, 10:20], W_iou[:, 20:30], W_f]
    biases = [b_iou[0:10], b_iou[10:20], b_iou[20:30], b_f]
    w4 = jnp.concatenate(
        [jnp.pad(g, ((0, 0), (0, 6))) for g in gates], axis=1)
    b4 = jnp.concatenate([jnp.pad(b, (0, 6)) for b in biases])
    u_gates = [U_iou[:, 0:10], U_iou[:, 10:20], U_iou[:, 20:30], U_f]
    uit, uot, uut, uft = [_pad16(u.T) for u in u_gates]
    return w4.T, b4.reshape(64, 1), uit, uot, uut, uft


def kernel(features, node_order, adjacency_list, edge_order, pe, emb_table,
           sort_table, W_iou, b_iou, U_iou, W_f, b_f, U_f):
    perm = jnp.asarray(_PERM)
    tokf, sortf, pef = _make_sc_gather()(
        perm, features[:, 0], features[:, 1], features[:, 2],
        jnp.pad(emb_table, ((0, 0), (0, 6))),
        jnp.pad(sort_table, ((0, 0), (0, 6))), pe)
    w4t, b4, uit, uot, uut, uft = _prep_weights(
        W_iou, b_iou, U_iou, W_f, b_f, U_f)
    h_roots_t = _tc_call(tokf, sortf, pef, w4t, b4, uit, uot, uut, uft)
    return h_roots_t[0:_TD].T


# speedup vs baseline: 29.4040x; 1.0214x over previous
import functools
import numpy as np
import jax
import jax.numpy as jnp
from jax import lax
from jax.experimental import pallas as pl
from jax.experimental.pallas import tpu as pltpu
from jax.experimental.pallas import tpu_sc as plsc

_DEPTH = 9
_NT = 64
_TD = 10
_NPT = 2 ** (_DEPTH + 1) - 1
_N = _NT * _NPT
_PAD_N = 65536
_NC, _NS = 2, 16
_NW = _NC * _NS
_ROWS_PER_W = _PAD_N // _NW
_CH = 1024


def _build_perm():
    sizes = [2 ** (_DEPTH - l) for l in range(_DEPTH + 1)]
    offs = np.concatenate([[0], np.cumsum(sizes)])
    tree = np.arange(_NT, dtype=np.int64)
    pos = np.zeros(_NT, dtype=np.int64)
    blocks = [None] * (_DEPTH + 1)
    blocks[_DEPTH] = tree * _NPT + offs[_DEPTH] + pos
    for l in range(_DEPTH, 0, -1):
        tree = np.concatenate([tree, tree])
        pos = np.concatenate([2 * pos, 2 * pos + 1])
        blocks[l - 1] = tree * _NPT + offs[l - 1] + pos
    perm = np.concatenate([blocks[l] for l in range(_DEPTH + 1)])
    perm = np.concatenate([perm, np.zeros(_PAD_N - _N, dtype=np.int64)])
    return perm.astype(np.int32)


_PERM = _build_perm()
_LV_N = [_NT * 2 ** (_DEPTH - l) for l in range(_DEPTH + 1)]


def _sc_gather_body(perm_hbm, tok_hbm, sort_hbm, pos_hbm,
                    emb_hbm, stab_hbm, pe_hbm,
                    otok, osort, ope,
                    pidx_v, ti_v, si_v, pi_v, rt_v, rs_v, rp_v, sem):
    wid = lax.axis_index("s") * _NC + lax.axis_index("c")
    for k in range(_ROWS_PER_W // _CH):
        b = wid * _ROWS_PER_W + k * _CH
        rows = pl.ds(b, _CH)
        pltpu.sync_copy(perm_hbm.at[rows], pidx_v)
        c1 = pltpu.async_copy(tok_hbm.at[pidx_v], ti_v, sem)
        c2 = pltpu.async_copy(sort_hbm.at[pidx_v], si_v, sem)
        c3 = pltpu.async_copy(pos_hbm.at[pidx_v], pi_v, sem)
        c1.wait(); c2.wait(); c3.wait()
        c1 = pltpu.async_copy(emb_hbm.at[ti_v], rt_v, sem)
        c2 = pltpu.async_copy(stab_hbm.at[si_v], rs_v, sem)
        c3 = pltpu.async_copy(pe_hbm.at[pi_v], rp_v, sem)
        c1.wait(); c2.wait(); c3.wait()
        c1 = pltpu.async_copy(rt_v, otok.at[rows], sem)
        c2 = pltpu.async_copy(rs_v, osort.at[rows], sem)
        c3 = pltpu.async_copy(rp_v, ope.at[rows], sem)
        c1.wait(); c2.wait(); c3.wait()


@functools.cache
def _make_sc_gather():
    return functools.partial(
        pl.kernel,
        out_type=[
            jax.ShapeDtypeStruct((_PAD_N, 16), jnp.float32),
            jax.ShapeDtypeStruct((_PAD_N, 16), jnp.float32),
            jax.ShapeDtypeStruct((_PAD_N, 32), jnp.float32),
        ],
        mesh=plsc.VectorSubcoreMesh(core_axis_name="c", subcore_axis_name="s"),
        scratch_types=[
            pltpu.VMEM((_CH,), jnp.int32),
            pltpu.VMEM((_CH,), jnp.int32),
            pltpu.VMEM((_CH,), jnp.int32),
            pltpu.VMEM((_CH,), jnp.int32),
            pltpu.VMEM((_CH, 16), jnp.float32),
            pltpu.VMEM((_CH, 16), jnp.float32),
            pltpu.VMEM((_CH, 32), jnp.float32),
            pltpu.SemaphoreType.DMA,
        ],
        compiler_params=pltpu.CompilerParams(use_tc_tiling_on_sc=False),
    )(_sc_gather_body)


def _sig(x):
    return 0.5 * jnp.tanh(0.5 * x) + 0.5


_PROJ_CH = 8192
_NPROJ = _PAD_N // _PROJ_CH


def _tc_body(tok_ref, sort_ref, pe_ref, wt_ref, b_ref,
             ust_ref, uft_ref, out_ref, iouf_ref):
    g = pl.program_id(0)

    @pl.when(g < _NPROJ)
    def _proj():
        x = jnp.concatenate([tok_ref[:, 0:_TD], sort_ref[:, 0:_TD],
                             pe_ref[:]], axis=1)
        res_t = jax.lax.dot_general(
            wt_ref[:], x, (((1,), (1,)), ((), ())),
            preferred_element_type=jnp.float32)
        iouf_ref[:, pl.ds(g * _PROJ_CH, _PROJ_CH)] = res_t + b_ref[:]

    @pl.when(g == _NPROJ)
    def _walk():
        n = _LV_N[0]
        i = _sig(iouf_ref[0:16, 0:n])
        o = _sig(iouf_ref[16:32, 0:n])
        u = jnp.tanh(iouf_ref[32:48, 0:n])
        c = i * u
        h = o * jnp.tanh(c)
        off = n
        for l in range(1, _DEPTH + 1):
            n = _LV_N[l]
            hs = h[:, :n] + h[:, n:]
            uh = jnp.dot(ust_ref[:], hs, preferred_element_type=jnp.float32)
            i = _sig(iouf_ref[0:16, off:off + n] + uh[0:16])
            o = _sig(iouf_ref[16:32, off:off + n] + uh[16:32])
            u = jnp.tanh(iouf_ref[32:48, off:off + n] + uh[32:48])
            fp = iouf_ref[48:64, off:off + n]
            f = _sig(jnp.concatenate([fp, fp], axis=1) +
                     jnp.dot(uft_ref[:], h,
                             preferred_element_type=jnp.float32))
            fc = f * c
            c = i * u + fc[:, :n] + fc[:, n:]
            h = o * jnp.tanh(c)
            off += n
        out_ref[:] = h


def _pad16(m):
    return jnp.pad(m, ((0, 16 - m.shape[0]), (0, 16 - m.shape[1])))


def _tc_call(tokf, sortf, pef, w4t, b4, ust, uft, interpret=False):
    return pl.pallas_call(
        _tc_body,
        grid=(_NPROJ + 1,),
        in_specs=[
            pl.BlockSpec((_PROJ_CH, 16),
                         lambda g: (jnp.minimum(g, _NPROJ - 1), 0)),
            pl.BlockSpec((_PROJ_CH, 16),
                         lambda g: (jnp.minimum(g, _NPROJ - 1), 0)),
            pl.BlockSpec((_PROJ_CH, 32),
                         lambda g: (jnp.minimum(g, _NPROJ - 1), 0)),
            pl.BlockSpec((64, 52), lambda g: (0, 0)),
            pl.BlockSpec((64, 1), lambda g: (0, 0)),
            pl.BlockSpec((48, 16), lambda g: (0, 0)),
            pl.BlockSpec((16, 16), lambda g: (0, 0)),
        ],
        out_specs=pl.BlockSpec((16, _NT), lambda g: (0, 0)),
        out_shape=jax.ShapeDtypeStruct((16, _NT), jnp.float32),
        scratch_shapes=[pltpu.VMEM((64, _PAD_N), jnp.float32)],
        interpret=interpret,
    )(tokf, sortf, pef, w4t, b4, ust, uft)


def _prep_weights(W_iou, b_iou, U_iou, W_f, b_f, U_f):
    gates = [W_iou[:, 0:10], W_iou[:, 10:20], W_iou[:, 20:30], W_f]
    biases = [b_iou[0:10], b_iou[10:20], b_iou[20:30], b_f]
    w4 = jnp.concatenate(
        [jnp.pad(g, ((0, 0), (0, 6))) for g in gates], axis=1)
    b4 = jnp.concatenate([jnp.pad(b, (0, 6)) for b in biases])
    u_gates = [U_iou[:, 0:10], U_iou[:, 10:20], U_iou[:, 20:30], U_f]
    uit, uot, uut, uft = [_pad16(u.T) for u in u_gates]
    ust = jnp.concatenate([uit, uot, uut], axis=0)
    return w4.T, b4.reshape(64, 1), ust, uft


def kernel(features, node_order, adjacency_list, edge_order, pe, emb_table,
           sort_table, W_iou, b_iou, U_iou, W_f, b_f, U_f):
    perm = jnp.asarray(_PERM)
    tokf, sortf, pef = _make_sc_gather()(
        perm, features[:, 0], features[:, 1], features[:, 2],
        jnp.pad(emb_table, ((0, 0), (0, 6))),
        jnp.pad(sort_table, ((0, 0), (0, 6))), pe)
    w4t, b4, ust, uft = _prep_weights(W_iou, b_iou, U_iou, W_f, b_f, U_f)
    h_roots_t = _tc_call(tokf, sortf, pef, w4t, b4, ust, uft)
    return h_roots_t[0:_TD].T


# speedup vs baseline: 32.2572x; 1.0970x over previous
import functools
import numpy as np
import jax
import jax.numpy as jnp
from jax import lax
from jax.experimental import pallas as pl
from jax.experimental.pallas import tpu as pltpu
from jax.experimental.pallas import tpu_sc as plsc

_DEPTH = 9
_NT = 64
_TD = 10
_NPT = 2 ** (_DEPTH + 1) - 1
_N = _NT * _NPT
_PAD_N = 65536
_NC, _NS = 2, 16
_NW = _NC * _NS
_ROWS_PER_W = _PAD_N // _NW
_CH = 1024


def _build_perm():
    sizes = [2 ** (_DEPTH - l) for l in range(_DEPTH + 1)]
    offs = np.concatenate([[0], np.cumsum(sizes)])
    tree = np.arange(_NT, dtype=np.int64)
    pos = np.zeros(_NT, dtype=np.int64)
    blocks = [None] * (_DEPTH + 1)
    blocks[_DEPTH] = tree * _NPT + offs[_DEPTH] + pos
    for l in range(_DEPTH, 0, -1):
        tree = np.concatenate([tree, tree])
        pos = np.concatenate([2 * pos, 2 * pos + 1])
        blocks[l - 1] = tree * _NPT + offs[l - 1] + pos
    perm = np.concatenate([blocks[l] for l in range(_DEPTH + 1)])
    perm = np.concatenate([perm, np.zeros(_PAD_N - _N, dtype=np.int64)])
    return perm.astype(np.int32)


_PERM = _build_perm()
_LV_N = [_NT * 2 ** (_DEPTH - l) for l in range(_DEPTH + 1)]


def _sc_gather_body(perm_hbm, tok_hbm, sort_hbm, pos_hbm,
                    ctab_hbm, pe_hbm,
                    otok, osort, ope,
                    pidx_v, ti_v, si_v, pi_v, rt_v, rs_v, rp_v, sem):
    wid = lax.axis_index("s") * _NC + lax.axis_index("c")
    for k in range(_ROWS_PER_W // _CH):
        b = wid * _ROWS_PER_W + k * _CH
        rows = pl.ds(b, _CH)
        pltpu.sync_copy(perm_hbm.at[rows], pidx_v)
        c1 = pltpu.async_copy(tok_hbm.at[pidx_v], ti_v, sem)
        c2 = pltpu.async_copy(sort_hbm.at[pidx_v], si_v, sem)
        c3 = pltpu.async_copy(pos_hbm.at[pidx_v], pi_v, sem)
        c1.wait(); c2.wait(); c3.wait()
        c1 = pltpu.async_copy(ctab_hbm.at[ti_v], rt_v, sem)
        c2 = pltpu.async_copy(ctab_hbm.at[si_v], rs_v, sem)
        c3 = pltpu.async_copy(pe_hbm.at[pi_v], rp_v, sem)
        c1.wait(); c2.wait(); c3.wait()
        c1 = pltpu.async_copy(rt_v, otok.at[rows], sem)
        c2 = pltpu.async_copy(rs_v, osort.at[rows], sem)
        c3 = pltpu.async_copy(rp_v, ope.at[rows], sem)
        c1.wait(); c2.wait(); c3.wait()


@functools.cache
def _make_sc_gather():
    return functools.partial(
        pl.kernel,
        out_type=[
            jax.ShapeDtypeStruct((_PAD_N, 16), jnp.float32),
            jax.ShapeDtypeStruct((_PAD_N, 16), jnp.float32),
            jax.ShapeDtypeStruct((_PAD_N, 32), jnp.float32),
        ],
        mesh=plsc.VectorSubcoreMesh(core_axis_name="c", subcore_axis_name="s"),
        scratch_types=[
            pltpu.VMEM((_CH,), jnp.int32),
            pltpu.VMEM((_CH,), jnp.int32),
            pltpu.VMEM((_CH,), jnp.int32),
            pltpu.VMEM((_CH,), jnp.int32),
            pltpu.VMEM((_CH, 16), jnp.float32),
            pltpu.VMEM((_CH, 16), jnp.float32),
            pltpu.VMEM((_CH, 32), jnp.float32),
            pltpu.SemaphoreType.DMA,
        ],
        compiler_params=pltpu.CompilerParams(use_tc_tiling_on_sc=False),
    )(_sc_gather_body)


def _sig(x):
    return 0.5 * jnp.tanh(0.5 * x) + 0.5


_PROJ_CH = 8192
_NPROJ = _PAD_N // _PROJ_CH


def _tc_body(tok_ref, sort_ref, pe_ref, wt_ref, b_ref,
             ust_ref, uft_ref, out_ref, iouf_ref):
    g = pl.program_id(0)

    @pl.when(g < _NPROJ)
    def _proj():
        x = jnp.concatenate([tok_ref[:, 0:_TD], sort_ref[:, 0:_TD],
                             pe_ref[:]], axis=1)
        res_t = jax.lax.dot_general(
            wt_ref[:], x, (((0,), (1,)), ((), ())),
            preferred_element_type=jnp.float32)
        iouf_ref[:, pl.ds(g * _PROJ_CH, _PROJ_CH)] = res_t + b_ref[:]

    @pl.when(g == _NPROJ)
    def _walk():
        n = _LV_N[0]
        i = _sig(iouf_ref[0:16, 0:n])
        o = _sig(iouf_ref[16:32, 0:n])
        u = jnp.tanh(iouf_ref[32:48, 0:n])
        c = i * u
        h = o * jnp.tanh(c)
        off = n
        for l in range(1, _DEPTH + 1):
            n = _LV_N[l]
            hs = h[:, :n] + h[:, n:]
            uh = jax.lax.dot_general(
                ust_ref[:], hs, (((0,), (0,)), ((), ())),
                preferred_element_type=jnp.float32)
            i = _sig(iouf_ref[0:16, off:off + n] + uh[0:16])
            o = _sig(iouf_ref[16:32, off:off + n] + uh[16:32])
            u = jnp.tanh(iouf_ref[32:48, off:off + n] + uh[32:48])
            fp = iouf_ref[48:64, off:off + n]
            f = _sig(jnp.concatenate([fp, fp], axis=1) +
                     jax.lax.dot_general(
                         uft_ref[:], h, (((0,), (0,)), ((), ())),
                         preferred_element_type=jnp.float32))
            fc = f * c
            c = i * u + fc[:, :n] + fc[:, n:]
            h = o * jnp.tanh(c)
            off += n
        out_ref[:] = jnp.transpose(h)[:, 0:_TD]


def _pad16(m):
    return jnp.pad(m, ((0, 16 - m.shape[0]), (0, 16 - m.shape[1])))


def _tc_call(tokf, sortf, pef, w4t, b4, ust, uft, interpret=False):
    return pl.pallas_call(
        _tc_body,
        grid=(_NPROJ + 1,),
        in_specs=[
            pl.BlockSpec((_PROJ_CH, 16),
                         lambda g: (jnp.minimum(g, _NPROJ - 1), 0)),
            pl.BlockSpec((_PROJ_CH, 16),
                         lambda g: (jnp.minimum(g, _NPROJ - 1), 0)),
            pl.BlockSpec((_PROJ_CH, 32),
                         lambda g: (jnp.minimum(g, _NPROJ - 1), 0)),
            pl.BlockSpec((52, 64), lambda g: (0, 0)),
            pl.BlockSpec((64, 1), lambda g: (0, 0)),
            pl.BlockSpec((16, 48), lambda g: (0, 0)),
            pl.BlockSpec((16, 16), lambda g: (0, 0)),
        ],
        out_specs=pl.BlockSpec((_NT, _TD), lambda g: (0, 0)),
        out_shape=jax.ShapeDtypeStruct((_NT, _TD), jnp.float32),
        scratch_shapes=[pltpu.VMEM((64, _PAD_N), jnp.float32)],
        interpret=interpret,
    )(tokf, sortf, pef, w4t, b4, ust, uft)


def _prep_weights(W_iou, b_iou, U_iou, W_f, b_f, U_f):
    gates = [W_iou[:, 0:10], W_iou[:, 10:20], W_iou[:, 20:30], W_f]
    biases = [b_iou[0:10], b_iou[10:20], b_iou[20:30], b_f]
    w4 = jnp.concatenate(
        [jnp.pad(g, ((0, 0), (0, 6))) for g in gates], axis=1)
    b4 = jnp.concatenate([jnp.pad(b, (0, 6)) for b in biases])
    u_gates = [U_iou[:, 0:10], U_iou[:, 10:20], U_iou[:, 20:30], U_f]
    uit, uot, uut, uft = [_pad16(u) for u in u_gates]
    ust = jnp.concatenate([uit, uot, uut], axis=1)
    return w4, b4.reshape(64, 1), ust, uft


def kernel(features, node_order, adjacency_list, edge_order, pe, emb_table,
           sort_table, W_iou, b_iou, U_iou, W_f, b_f, U_f):
    perm = jnp.asarray(_PERM)
    ctab = jnp.pad(jnp.concatenate([emb_table, sort_table], axis=0),
                   ((0, 0), (0, 6)))
    tokf, sortf, pef = _make_sc_gather()(
        perm, features[:, 0], features[:, 1] + emb_table.shape[0],
        features[:, 2], ctab, pe)
    w4, b4, ust, uft = _prep_weights(W_iou, b_iou, U_iou, W_f, b_f, U_f)
    return _tc_call(tokf, sortf, pef, w4, b4, ust, uft)
